# baseline (speedup 1.0000x reference)
import functools
import jax
import jax.numpy as jnp
from jax import lax
from jax.experimental import pallas as pl
from jax.experimental.pallas import tpu as pltpu
from jax.experimental.pallas import tpu_sc as plsc

H = 2
BR = 512
NC = 2
NS = 16
NW = NC * NS
NP = 51200
WIN = NP // NW
GRP = 64
NGRP = WIN // GRP
EC = 128


def _proj_body(x_ref, w_ref, b_ref, q_ref, kv_ref):
    m = q_ref.shape[1]
    o = jnp.dot(x_ref[...], w_ref[...],
                preferred_element_type=jnp.float32) + b_ref[...]
    q_ref[...] = o[:, :m]
    kv_ref[...] = o[:, m:]


def _proj3(x, w, b):
    np_, k = x.shape
    m3 = w.shape[1]
    m = m3 // 3
    return pl.pallas_call(
        _proj_body,
        grid=(np_ // BR,),
        in_specs=[
            pl.BlockSpec((BR, k), lambda i: (i, 0)),
            pl.BlockSpec((k, m3), lambda i: (0, 0)),
            pl.BlockSpec((1, m3), lambda i: (0, 0)),
        ],
        out_specs=[pl.BlockSpec((BR, m), lambda i: (i, 0)),
                   pl.BlockSpec((BR, 2 * m), lambda i: (i, 0))],
        out_shape=[jax.ShapeDtypeStruct((np_, m), jnp.float32),
                   jax.ShapeDtypeStruct((np_, 2 * m), jnp.float32)],
    )(x, w, b.reshape(1, m3))


def _epi_body(raw_ref, s0_ref, s1_ref, xs_ref, wa_ref, ba_ref, beta_ref,
              o_ref, *, d):
    s0, s1 = s0_ref[...], s1_ref[...]
    raw = raw_ref[...]
    m0 = jnp.where(s0 > 0, raw[:, :d] / (s0 + 1e-16), 0.0)
    m1 = jnp.where(s1 > 0, raw[:, d:] / (s1 + 1e-16), 0.0)
    msg = jnp.concatenate([m0, m1], axis=1)
    out = jnp.dot(jax.nn.gelu(msg), wa_ref[...],
                  preferred_element_type=jnp.float32) + ba_ref[...]
    beta = beta_ref[0, 0]
    o_ref[...] = jnp.maximum(beta * out + (1.0 - beta) * xs_ref[...], 0.0)


def _conv_epi(raw, s0, s1, x_skip, wa, ba, beta):
    np_, cout = raw.shape
    return pl.pallas_call(
        functools.partial(_epi_body, d=cout // H),
        grid=(np_ // BR,),
        in_specs=[
            pl.BlockSpec((BR, cout), lambda i: (i, 0)),
            pl.BlockSpec((BR, 1), lambda i: (i, 0)),
            pl.BlockSpec((BR, 1), lambda i: (i, 0)),
            pl.BlockSpec((BR, cout), lambda i: (i, 0)),
            pl.BlockSpec((cout, cout), lambda i: (0, 0)),
            pl.BlockSpec((1, cout), lambda i: (0, 0)),
            pl.BlockSpec((1, 1), lambda i: (0, 0), memory_space=pltpu.SMEM),
        ],
        out_specs=pl.BlockSpec((BR, cout), lambda i: (i, 0)),
        out_shape=jax.ShapeDtypeStruct((np_, cout), jnp.float32),
    )(raw, s0.reshape(np_, 1), s1.reshape(np_, 1), x_skip,
      wa, ba.reshape(1, cout), beta.reshape(1, 1))


def _gate_body(xg_ref, w1_ref, b1_ref, w2_ref, b2_ref, o_ref, *, n_real):
    i = pl.program_id(0)
    h = jnp.maximum(jnp.dot(xg_ref[...], w1_ref[...],
                            preferred_element_type=jnp.float32) + b1_ref[...], 0.0)
    w = jnp.dot(h, w2_ref[...], preferred_element_type=jnp.float32) + b2_ref[...]
    row = i * BR + jax.lax.broadcasted_iota(jnp.int32, w.shape, 0)
    o_ref[...] = jnp.where(row < n_real, jnp.exp(w), 0.0)


def _gate(xg, w1, b1, w2, b2, n_real):
    np_, k = xg.shape
    return pl.pallas_call(
        functools.partial(_gate_body, n_real=n_real),
        grid=(np_ // BR,),
        in_specs=[
            pl.BlockSpec((BR, k), lambda i: (i, 0)),
            pl.BlockSpec((k, 32), lambda i: (0, 0)),
            pl.BlockSpec((1, 32), lambda i: (0, 0)),
            pl.BlockSpec((32, 1), lambda i: (0, 0)),
            pl.BlockSpec((1, 1), lambda i: (0, 0)),
        ],
        out_specs=pl.BlockSpec((BR, 1), lambda i: (i, 0)),
        out_shape=jax.ShapeDtypeStruct((np_, 1), jnp.float32),
    )(xg, w1, b1.reshape(1, 32), w2, b2.reshape(1, 1))


def _pool_body(h3_ref, ew_ref, b_ref, o_ref, *, ng):
    i = pl.program_id(0)

    @pl.when(i == 0)
    def _():
        o_ref[...] = jnp.zeros_like(o_ref)

    ew = ew_ref[...]
    feats = jnp.concatenate([h3_ref[...] * ew, ew], axis=1)
    oh = (b_ref[...] == jax.lax.broadcasted_iota(jnp.int32, (BR, ng), 1)
          ).astype(jnp.float32)
    o_ref[...] += jax.lax.dot_general(
        oh, feats, (((0,), (0,)), ((), ())),
        preferred_element_type=jnp.float32)


def _pool(h3, ew, batch2d, ng):
    np_, cout = h3.shape
    return pl.pallas_call(
        functools.partial(_pool_body, ng=ng),
        grid=(np_ // BR,),
        in_specs=[
            pl.BlockSpec((BR, cout), lambda i: (i, 0)),
            pl.BlockSpec((BR, 1), lambda i: (i, 0)),
            pl.BlockSpec((BR, 1), lambda i: (i, 0)),
        ],
        out_specs=pl.BlockSpec((ng, cout + 1), lambda i: (0, 0)),
        out_shape=jax.ShapeDtypeStruct((ng, cout + 1), jnp.float32),
    )(h3, ew, batch2d)


def _final_body(p_ref, wm1_ref, bm1_ref, wm2_ref, bm2_ref, o_ref, *, cout):
    p = p_ref[...]
    g = p[:, :cout] / (p[:, cout:cout + 1] + 1e-16)
    h = jnp.maximum(jnp.dot(g, wm1_ref[...],
                            preferred_element_type=jnp.float32) + bm1_ref[...], 0.0)
    o_ref[...] = jnp.dot(h, wm2_ref[...],
                         preferred_element_type=jnp.float32) + bm2_ref[...]


def _final(pooled, wm1, bm1, wm2, bm2):
    ng, c1 = pooled.shape
    cout = c1 - 1
    hid = wm1.shape[1]
    out = wm2.shape[1]
    return pl.pallas_call(
        functools.partial(_final_body, cout=cout),
        in_specs=[
            pl.BlockSpec((ng, c1), lambda: (0, 0)),
            pl.BlockSpec((cout, hid), lambda: (0, 0)),
            pl.BlockSpec((1, hid), lambda: (0, 0)),
            pl.BlockSpec((hid, out), lambda: (0, 0)),
            pl.BlockSpec((1, out), lambda: (0, 0)),
        ],
        out_specs=pl.BlockSpec((ng, out), lambda: (0, 0)),
        out_shape=jax.ShapeDtypeStruct((ng, out), jnp.float32),
    )(pooled, wm1, bm1.reshape(1, hid), wm2, bm2.reshape(1, out))


def _zero16(ref, r, c0, nchunks):
    z = jnp.zeros((16,), jnp.float32)
    for c in range(nchunks):
        ref[r, pl.ds(c0 + 16 * c, 16)] = z


def _sload(ref, i):
    return ref[pl.ds(i, 16)][0]


def _edge_sc_body(q_hbm, kv_hbm, src_hbm, dst_hbm, rp_hbm,
                  raw_hbm, s0_hbm, s1_hbm,
                  rp_v, q_g, kvrA, kvrB, srcA, srcB, dstA, dstB,
                  st, s0_st, s1_st, s0_grp, s1_grp, e0_buf, e1_buf,
                  semA, semB):
    wid = lax.axis_index("s") * NC + lax.axis_index("c")
    v0 = wid * WIN
    pltpu.sync_copy(rp_hbm.at[pl.ds(v0, WIN + 16)], rp_v)

    def group_body(g, _):
        vg = v0 + g * GRP
        pltpu.sync_copy(q_hbm.at[pl.ds(vg, GRP)], q_g)

        def zrow(r, _):
            _zero16(st, r, 0, 8)
            s0_st[r, :] = jnp.zeros((16,), jnp.float32)
            s1_st[r, :] = jnp.zeros((16,), jnp.float32)
            return 0
        lax.fori_loop(0, GRP, zrow, 0)

        e_start = _sload(rp_v, g * GRP)
        e_end = _sload(rp_v, (g + 1) * GRP)
        ca0 = (e_start // 8) * 8
        nch = (e_end - ca0 + EC - 1) // EC

        def load_and_issue(c, src_c, dst_c, kvr, sem):
            ca = ca0 + c * EC
            pltpu.sync_copy(src_hbm.at[pl.ds(ca, EC)], src_c)
            pltpu.sync_copy(dst_hbm.at[pl.ds(ca, EC + 16)], dst_c)
            pltpu.make_async_copy(kv_hbm.at[src_c], kvr, sem).start()

        def compute(c, dst_c, kvr):
            ca = ca0 + c * EC
            lo = jnp.maximum(e_start - ca, 0)
            hi = jnp.minimum(e_end - ca, EC)

            def grp16(t, _):
                j0 = t * 16
                nvec = jnp.clip(dst_c[pl.ds(j0, 16)] - vg, 0, GRP - 1)
                jvec = lax.iota(jnp.int32, 16) + j0
                mask = jnp.logical_and(jvec >= lo, jvec < hi)
                acc0 = jnp.zeros((16,), jnp.float32)
                acc1 = jnp.zeros((16,), jnp.float32)
                for p in range(64):
                    acc0 += (plsc.load_gather(q_g, [nvec, jnp.full((16,), p, jnp.int32)])
                             * plsc.load_gather(kvr, [jvec, jnp.full((16,), p, jnp.int32)]))
                    acc1 += (plsc.load_gather(q_g, [nvec, jnp.full((16,), 64 + p, jnp.int32)])
                             * plsc.load_gather(kvr, [jvec, jnp.full((16,), 64 + p, jnp.int32)]))
                e0 = jnp.exp(acc0)
                e1 = jnp.exp(acc1)
                for p in range(64):
                    v0 = plsc.load_gather(kvr, [jvec, jnp.full((16,), 128 + p, jnp.int32)])
                    plsc.addupdate_scatter(st, [nvec, jnp.full((16,), p, jnp.int32)],
                                           e0 * v0, mask=mask)
                    v1 = plsc.load_gather(kvr, [jvec, jnp.full((16,), 192 + p, jnp.int32)])
                    plsc.addupdate_scatter(st, [nvec, jnp.full((16,), 64 + p, jnp.int32)],
                                           e1 * v1, mask=mask)
                zc16 = jnp.zeros((16,), jnp.int32)
                plsc.addupdate_scatter(s0_st, [nvec, zc16], e0, mask=mask)
                plsc.addupdate_scatter(s1_st, [nvec, zc16], e1, mask=mask)
                return 0
            lax.fori_loop(0, EC // 16, grp16, 0)

        def waitA():
            pltpu.make_async_copy(kv_hbm.at[srcA], kvrA, semA).wait()

        def waitB():
            pltpu.make_async_copy(kv_hbm.at[srcB], kvrB, semB).wait()

        def chunk_body(c, _):
            load_and_issue(c, srcA, dstA, kvrA, semA)
            waitA()
            compute(c, dstA, kvrA)
            return 0
        lax.fori_loop(0, nch, chunk_body, 0)

        rows16 = lax.iota(jnp.int32, 16)
        zc = jnp.zeros((16,), jnp.int32)
        for r in range(4):
            idx = rows16 + 16 * r
            s0_grp[pl.ds(16 * r, 16)] = plsc.load_gather(s0_st, [idx, zc])
            s1_grp[pl.ds(16 * r, 16)] = plsc.load_gather(s1_st, [idx, zc])
        pltpu.sync_copy(st, raw_hbm.at[pl.ds(vg, GRP)])
        pltpu.sync_copy(s0_grp, s0_hbm.at[pl.ds(vg, GRP)])
        pltpu.sync_copy(s1_grp, s1_hbm.at[pl.ds(vg, GRP)])
        return 0
    lax.fori_loop(0, NGRP, group_body, 0)


def _edge_phase(q, kv, src_s, dst_s, rowptr):
    mesh = plsc.VectorSubcoreMesh(core_axis_name="c", subcore_axis_name="s")
    f = pl.kernel(
        _edge_sc_body,
        mesh=mesh,
        compiler_params=pltpu.CompilerParams(needs_layout_passes=False),
        out_type=[
            jax.ShapeDtypeStruct((NP, H * 64), jnp.float32),
            jax.ShapeDtypeStruct((NP,), jnp.float32),
            jax.ShapeDtypeStruct((NP,), jnp.float32),
        ],
        scratch_types=[
            pltpu.VMEM((WIN + 16,), jnp.int32),
            pltpu.VMEM((GRP, H * 64), jnp.float32),
            pltpu.VMEM((EC, 2 * H * 64), jnp.float32),
            pltpu.VMEM((EC, 2 * H * 64), jnp.float32),
            pltpu.VMEM((EC,), jnp.int32),
            pltpu.VMEM((EC,), jnp.int32),
            pltpu.VMEM((EC + 16,), jnp.int32),
            pltpu.VMEM((EC + 16,), jnp.int32),
            pltpu.VMEM((GRP, H * 64), jnp.float32),
            pltpu.VMEM((GRP, 16), jnp.float32),
            pltpu.VMEM((GRP, 16), jnp.float32),
            pltpu.VMEM((GRP,), jnp.float32),
            pltpu.VMEM((GRP,), jnp.float32),
            pltpu.VMEM((EC + 16,), jnp.float32),
            pltpu.VMEM((EC + 16,), jnp.float32),
            pltpu.SemaphoreType.DMA,
            pltpu.SemaphoreType.DMA,
        ],
    )
    return f(q, kv, src_s, dst_s, rowptr)


def _fold_kv(p, cin):
    cout = p['Wk'].shape[1]
    d = cout // H
    wk = jnp.einsum('chd,hde->che', p['Wk'].reshape(cin, H, d), p['a_rel']).reshape(cin, cout)
    bk = jnp.einsum('hd,hde->he', p['bk'].reshape(H, d), p['a_rel']).reshape(cout)
    wv = jnp.einsum('chd,hde->che', p['Wv'].reshape(cin, H, d), p['m_rel']).reshape(cin, cout)
    bv = jnp.einsum('hd,hde->he', p['bv'].reshape(H, d), p['m_rel']).reshape(cout)
    return wk, bk, wv, bv


def _one_conv(x_in, src_s, dst_s, rowptr, p, use_skip):
    cin = x_in.shape[1]
    cout = p['Wk'].shape[1]
    d = cout // H
    wk, bk, wv, bv = _fold_kv(p, cin)
    scale = p['p_rel'] / jnp.sqrt(jnp.float32(d))
    wq = p['Wq'] * jnp.repeat(scale, d)[None, :]
    bq = p['bq'] * jnp.repeat(scale, d)
    wqkv = jnp.concatenate([wq, wk, wv], axis=1)
    bqkv = jnp.concatenate([bq, bk, bv])
    q, kv = _proj3(x_in, wqkv, bqkv)
    raw, s0, s1 = _edge_phase(q, kv, src_s, dst_s, rowptr)
    beta = jax.nn.sigmoid(p['skip']) if use_skip else jnp.float32(1.0)
    x_skip = x_in if use_skip else raw
    return _conv_epi(raw, s0, s1, x_skip, p['Wa'], p['ba'], beta)


def kernel(x, goal, edge_index, batch, params):
    n, f_in = x.shape
    ng = 64
    pad = NP - n
    xp = jnp.pad(x, ((0, pad), (0, 0)))

    src, dst = edge_index[0].astype(jnp.int32), edge_index[1].astype(jnp.int32)
    order = jnp.argsort(dst)
    src_s = jnp.pad(src[order], (0, 2 * EC))
    dst_s = jnp.pad(dst[order], (0, 2 * EC))
    rowptr = jnp.searchsorted(dst_s[:dst.shape[0]],
                              jnp.arange(NP + 16, dtype=jnp.int32)).astype(jnp.int32)

    gd = goal.shape[1]
    kcat = f_in + gd
    kpad = ((kcat + 7) // 8) * 8
    xg = jnp.pad(jnp.concatenate([x, goal], axis=1), ((0, pad), (0, kpad - kcat)))
    w1 = jnp.pad(params['W1'], ((0, kpad - kcat), (0, 0)))
    ew = _gate(xg, w1, params['b1'], params['W2'], params['b2'], n)

    h1 = _one_conv(xp, src_s, dst_s, rowptr, params['c1'], False)
    h2 = _one_conv(h1, src_s, dst_s, rowptr, params['c2'], True)
    h3 = _one_conv(h2, src_s, dst_s, rowptr, params['c3'], True)

    batch2d = jnp.pad(batch.astype(jnp.int32), (0, pad)).reshape(NP, 1)
    pooled = _pool(h3, ew, batch2d, ng)
    return _final(pooled, params['Wm1'], params['bm1'], params['Wm2'], params['bm2'])


# speedup vs baseline: 3.2079x; 3.2079x over previous
import functools
import jax
import jax.numpy as jnp
from jax import lax
from jax.experimental import pallas as pl
from jax.experimental.pallas import tpu as pltpu
from jax.experimental.pallas import tpu_sc as plsc

H = 2
BR = 512
NC = 2
NS = 16
NW = NC * NS
NP = 51200
WIN = NP // NW
GRP = 64
NGRP = WIN // GRP
EC = 128


def _proj_body(x_ref, w_ref, b_ref, q_ref, kv_ref):
    m = q_ref.shape[1]
    o = jnp.dot(x_ref[...], w_ref[...],
                preferred_element_type=jnp.float32) + b_ref[...]
    q_ref[...] = o[:, :m]
    kv_ref[...] = o[:, m:]


def _proj3(x, w, b):
    np_, k = x.shape
    m3 = w.shape[1]
    m = m3 // 3
    return pl.pallas_call(
        _proj_body,
        grid=(np_ // BR,),
        in_specs=[
            pl.BlockSpec((BR, k), lambda i: (i, 0)),
            pl.BlockSpec((k, m3), lambda i: (0, 0)),
            pl.BlockSpec((1, m3), lambda i: (0, 0)),
        ],
        out_specs=[pl.BlockSpec((BR, m), lambda i: (i, 0)),
                   pl.BlockSpec((BR, 2 * m), lambda i: (i, 0))],
        out_shape=[jax.ShapeDtypeStruct((np_, m), jnp.float32),
                   jax.ShapeDtypeStruct((np_, 2 * m), jnp.float32)],
    )(x, w, b.reshape(1, m3))


def _epi_body(raw_ref, s0_ref, s1_ref, xs_ref, wa_ref, ba_ref, beta_ref,
              o_ref, *, d):
    s0, s1 = s0_ref[...], s1_ref[...]
    raw = raw_ref[...]
    m0 = jnp.where(s0 > 0, raw[:, :d] / (s0 + 1e-16), 0.0)
    m1 = jnp.where(s1 > 0, raw[:, d:] / (s1 + 1e-16), 0.0)
    msg = jnp.concatenate([m0, m1], axis=1)
    out = jnp.dot(jax.nn.gelu(msg), wa_ref[...],
                  preferred_element_type=jnp.float32) + ba_ref[...]
    beta = beta_ref[0, 0]
    o_ref[...] = jnp.maximum(beta * out + (1.0 - beta) * xs_ref[...], 0.0)


def _conv_epi(raw, s0, s1, x_skip, wa, ba, beta):
    np_, cout = raw.shape
    return pl.pallas_call(
        functools.partial(_epi_body, d=cout // H),
        grid=(np_ // BR,),
        in_specs=[
            pl.BlockSpec((BR, cout), lambda i: (i, 0)),
            pl.BlockSpec((BR, 1), lambda i: (i, 0)),
            pl.BlockSpec((BR, 1), lambda i: (i, 0)),
            pl.BlockSpec((BR, cout), lambda i: (i, 0)),
            pl.BlockSpec((cout, cout), lambda i: (0, 0)),
            pl.BlockSpec((1, cout), lambda i: (0, 0)),
            pl.BlockSpec((1, 1), lambda i: (0, 0), memory_space=pltpu.SMEM),
        ],
        out_specs=pl.BlockSpec((BR, cout), lambda i: (i, 0)),
        out_shape=jax.ShapeDtypeStruct((np_, cout), jnp.float32),
    )(raw, s0.reshape(np_, 1), s1.reshape(np_, 1), x_skip,
      wa, ba.reshape(1, cout), beta.reshape(1, 1))


def _gate_body(xg_ref, w1_ref, b1_ref, w2_ref, b2_ref, o_ref, *, n_real):
    i = pl.program_id(0)
    h = jnp.maximum(jnp.dot(xg_ref[...], w1_ref[...],
                            preferred_element_type=jnp.float32) + b1_ref[...], 0.0)
    w = jnp.dot(h, w2_ref[...], preferred_element_type=jnp.float32) + b2_ref[...]
    row = i * BR + jax.lax.broadcasted_iota(jnp.int32, w.shape, 0)
    o_ref[...] = jnp.where(row < n_real, jnp.exp(w), 0.0)


def _gate(xg, w1, b1, w2, b2, n_real):
    np_, k = xg.shape
    return pl.pallas_call(
        functools.partial(_gate_body, n_real=n_real),
        grid=(np_ // BR,),
        in_specs=[
            pl.BlockSpec((BR, k), lambda i: (i, 0)),
            pl.BlockSpec((k, 32), lambda i: (0, 0)),
            pl.BlockSpec((1, 32), lambda i: (0, 0)),
            pl.BlockSpec((32, 1), lambda i: (0, 0)),
            pl.BlockSpec((1, 1), lambda i: (0, 0)),
        ],
        out_specs=pl.BlockSpec((BR, 1), lambda i: (i, 0)),
        out_shape=jax.ShapeDtypeStruct((np_, 1), jnp.float32),
    )(xg, w1, b1.reshape(1, 32), w2, b2.reshape(1, 1))


def _pool_body(h3_ref, ew_ref, b_ref, o_ref, *, ng):
    i = pl.program_id(0)

    @pl.when(i == 0)
    def _():
        o_ref[...] = jnp.zeros_like(o_ref)

    ew = ew_ref[...]
    feats = jnp.concatenate([h3_ref[...] * ew, ew], axis=1)
    oh = (b_ref[...] == jax.lax.broadcasted_iota(jnp.int32, (BR, ng), 1)
          ).astype(jnp.float32)
    o_ref[...] += jax.lax.dot_general(
        oh, feats, (((0,), (0,)), ((), ())),
        preferred_element_type=jnp.float32)


def _pool(h3, ew, batch2d, ng):
    np_, cout = h3.shape
    return pl.pallas_call(
        functools.partial(_pool_body, ng=ng),
        grid=(np_ // BR,),
        in_specs=[
            pl.BlockSpec((BR, cout), lambda i: (i, 0)),
            pl.BlockSpec((BR, 1), lambda i: (i, 0)),
            pl.BlockSpec((BR, 1), lambda i: (i, 0)),
        ],
        out_specs=pl.BlockSpec((ng, cout + 1), lambda i: (0, 0)),
        out_shape=jax.ShapeDtypeStruct((ng, cout + 1), jnp.float32),
    )(h3, ew, batch2d)


def _final_body(p_ref, wm1_ref, bm1_ref, wm2_ref, bm2_ref, o_ref, *, cout):
    p = p_ref[...]
    g = p[:, :cout] / (p[:, cout:cout + 1] + 1e-16)
    h = jnp.maximum(jnp.dot(g, wm1_ref[...],
                            preferred_element_type=jnp.float32) + bm1_ref[...], 0.0)
    o_ref[...] = jnp.dot(h, wm2_ref[...],
                         preferred_element_type=jnp.float32) + bm2_ref[...]


def _final(pooled, wm1, bm1, wm2, bm2):
    ng, c1 = pooled.shape
    cout = c1 - 1
    hid = wm1.shape[1]
    out = wm2.shape[1]
    return pl.pallas_call(
        functools.partial(_final_body, cout=cout),
        in_specs=[
            pl.BlockSpec((ng, c1), lambda: (0, 0)),
            pl.BlockSpec((cout, hid), lambda: (0, 0)),
            pl.BlockSpec((1, hid), lambda: (0, 0)),
            pl.BlockSpec((hid, out), lambda: (0, 0)),
            pl.BlockSpec((1, out), lambda: (0, 0)),
        ],
        out_specs=pl.BlockSpec((ng, out), lambda: (0, 0)),
        out_shape=jax.ShapeDtypeStruct((ng, out), jnp.float32),
    )(pooled, wm1, bm1.reshape(1, hid), wm2, bm2.reshape(1, out))


def _zero16(ref, r, c0, nchunks):
    z = jnp.zeros((16,), jnp.float32)
    for c in range(nchunks):
        ref[r, pl.ds(c0 + 16 * c, 16)] = z


def _sload(ref, i):
    return ref[pl.ds(i, 16)][0]


_GDN = jax.lax.GatherDimensionNumbers(
    offset_dims=(), collapsed_slice_dims=(0,), start_index_map=(0,))


def _bfly_sum(x):
    for k in (1, 2, 4, 8):
        perm = jnp.bitwise_xor(lax.iota(jnp.int32, 16), k)
        x = x + jax.lax.gather(x, perm[:, None], _GDN, (1,),
                               mode=jax.lax.GatherScatterMode.PROMISE_IN_BOUNDS)
    return x


def _edge_sc_body(q_hbm, kv_hbm, src_hbm, dst_hbm, rp_hbm,
                  raw_hbm, s0_hbm, s1_hbm,
                  rp_v, q_g, kvrA, kvrB, srcA, srcB, dstA, dstB,
                  st, s0_st, s1_st, s0_grp, s1_grp, e0_buf, e1_buf,
                  semA, semB):
    wid = lax.axis_index("s") * NC + lax.axis_index("c")
    v0 = wid * WIN
    pltpu.sync_copy(rp_hbm.at[pl.ds(v0, WIN + 16)], rp_v)

    def group_body(g, _):
        vg = v0 + g * GRP
        pltpu.sync_copy(q_hbm.at[pl.ds(vg, GRP)], q_g)

        def zrow(r, _):
            _zero16(st, r, 0, 8)
            s0_st[r, :] = jnp.zeros((16,), jnp.float32)
            s1_st[r, :] = jnp.zeros((16,), jnp.float32)
            return 0
        lax.fori_loop(0, GRP, zrow, 0)

        e_start = _sload(rp_v, g * GRP)
        e_end = _sload(rp_v, (g + 1) * GRP)
        ca0 = (e_start // 8) * 8
        nch = (e_end - ca0 + EC - 1) // EC

        def load_and_issue(c, src_c, dst_c, kvr, sem):
            ca = ca0 + c * EC
            pltpu.sync_copy(src_hbm.at[pl.ds(ca, EC)], src_c)
            pltpu.sync_copy(dst_hbm.at[pl.ds(ca, EC + 16)], dst_c)
            pltpu.make_async_copy(kv_hbm.at[src_c], kvr, sem).start()

        def compute(c, dst_c, kvr):
            ca = ca0 + c * EC
            lo = jnp.maximum(e_start - ca, 0)
            hi = jnp.minimum(e_end - ca, EC)

            def edge_body(j, _):
                n = _sload(dst_c, j) - vg
                acc0 = q_g[n, pl.ds(0, 16)] * kvr[j, pl.ds(0, 16)]
                acc1 = q_g[n, pl.ds(64, 16)] * kvr[j, pl.ds(64, 16)]
                for c4 in range(1, 4):
                    acc0 += q_g[n, pl.ds(16 * c4, 16)] * kvr[j, pl.ds(16 * c4, 16)]
                    acc1 += (q_g[n, pl.ds(64 + 16 * c4, 16)]
                             * kvr[j, pl.ds(64 + 16 * c4, 16)])
                e0 = jnp.exp(_bfly_sum(acc0))
                e1 = jnp.exp(_bfly_sum(acc1))
                for c4 in range(4):
                    plsc.addupdate(st.at[n, pl.ds(16 * c4, 16)],
                                   e0 * kvr[j, pl.ds(128 + 16 * c4, 16)])
                    plsc.addupdate(st.at[n, pl.ds(64 + 16 * c4, 16)],
                                   e1 * kvr[j, pl.ds(192 + 16 * c4, 16)])
                plsc.addupdate(s0_st.at[n], e0)
                plsc.addupdate(s1_st.at[n], e1)
                return 0
            lax.fori_loop(lo, hi, edge_body, 0)

        def waitA():
            pltpu.make_async_copy(kv_hbm.at[srcA], kvrA, semA).wait()

        def waitB():
            pltpu.make_async_copy(kv_hbm.at[srcB], kvrB, semB).wait()

        def chunk_body(c, _):
            load_and_issue(c, srcA, dstA, kvrA, semA)
            waitA()
            compute(c, dstA, kvrA)
            return 0
        lax.fori_loop(0, nch, chunk_body, 0)

        rows16 = lax.iota(jnp.int32, 16)
        zc = jnp.zeros((16,), jnp.int32)
        for r in range(4):
            idx = rows16 + 16 * r
            s0_grp[pl.ds(16 * r, 16)] = plsc.load_gather(s0_st, [idx, zc])
            s1_grp[pl.ds(16 * r, 16)] = plsc.load_gather(s1_st, [idx, zc])
        pltpu.sync_copy(st, raw_hbm.at[pl.ds(vg, GRP)])
        pltpu.sync_copy(s0_grp, s0_hbm.at[pl.ds(vg, GRP)])
        pltpu.sync_copy(s1_grp, s1_hbm.at[pl.ds(vg, GRP)])
        return 0
    lax.fori_loop(0, NGRP, group_body, 0)


def _edge_phase(q, kv, src_s, dst_s, rowptr):
    mesh = plsc.VectorSubcoreMesh(core_axis_name="c", subcore_axis_name="s")
    f = pl.kernel(
        _edge_sc_body,
        mesh=mesh,
        compiler_params=pltpu.CompilerParams(needs_layout_passes=False),
        out_type=[
            jax.ShapeDtypeStruct((NP, H * 64), jnp.float32),
            jax.ShapeDtypeStruct((NP,), jnp.float32),
            jax.ShapeDtypeStruct((NP,), jnp.float32),
        ],
        scratch_types=[
            pltpu.VMEM((WIN + 16,), jnp.int32),
            pltpu.VMEM((GRP, H * 64), jnp.float32),
            pltpu.VMEM((EC, 2 * H * 64), jnp.float32),
            pltpu.VMEM((EC, 2 * H * 64), jnp.float32),
            pltpu.VMEM((EC,), jnp.int32),
            pltpu.VMEM((EC,), jnp.int32),
            pltpu.VMEM((EC + 16,), jnp.int32),
            pltpu.VMEM((EC + 16,), jnp.int32),
            pltpu.VMEM((GRP, H * 64), jnp.float32),
            pltpu.VMEM((GRP, 16), jnp.float32),
            pltpu.VMEM((GRP, 16), jnp.float32),
            pltpu.VMEM((GRP,), jnp.float32),
            pltpu.VMEM((GRP,), jnp.float32),
            pltpu.VMEM((EC + 16,), jnp.float32),
            pltpu.VMEM((EC + 16,), jnp.float32),
            pltpu.SemaphoreType.DMA,
            pltpu.SemaphoreType.DMA,
        ],
    )
    return f(q, kv, src_s, dst_s, rowptr)


def _fold_kv(p, cin):
    cout = p['Wk'].shape[1]
    d = cout // H
    wk = jnp.einsum('chd,hde->che', p['Wk'].reshape(cin, H, d), p['a_rel']).reshape(cin, cout)
    bk = jnp.einsum('hd,hde->he', p['bk'].reshape(H, d), p['a_rel']).reshape(cout)
    wv = jnp.einsum('chd,hde->che', p['Wv'].reshape(cin, H, d), p['m_rel']).reshape(cin, cout)
    bv = jnp.einsum('hd,hde->he', p['bv'].reshape(H, d), p['m_rel']).reshape(cout)
    return wk, bk, wv, bv


def _one_conv(x_in, src_s, dst_s, rowptr, p, use_skip):
    cin = x_in.shape[1]
    cout = p['Wk'].shape[1]
    d = cout // H
    wk, bk, wv, bv = _fold_kv(p, cin)
    scale = p['p_rel'] / jnp.sqrt(jnp.float32(d))
    wq = p['Wq'] * jnp.repeat(scale, d)[None, :]
    bq = p['bq'] * jnp.repeat(scale, d)
    wqkv = jnp.concatenate([wq, wk, wv], axis=1)
    bqkv = jnp.concatenate([bq, bk, bv])
    q, kv = _proj3(x_in, wqkv, bqkv)
    raw, s0, s1 = _edge_phase(q, kv, src_s, dst_s, rowptr)
    beta = jax.nn.sigmoid(p['skip']) if use_skip else jnp.float32(1.0)
    x_skip = x_in if use_skip else raw
    return _conv_epi(raw, s0, s1, x_skip, p['Wa'], p['ba'], beta)


def kernel(x, goal, edge_index, batch, params):
    n, f_in = x.shape
    ng = 64
    pad = NP - n
    xp = jnp.pad(x, ((0, pad), (0, 0)))

    src, dst = edge_index[0].astype(jnp.int32), edge_index[1].astype(jnp.int32)
    order = jnp.argsort(dst)
    src_s = jnp.pad(src[order], (0, 2 * EC))
    dst_s = jnp.pad(dst[order], (0, 2 * EC))
    rowptr = jnp.searchsorted(dst_s[:dst.shape[0]],
                              jnp.arange(NP + 16, dtype=jnp.int32)).astype(jnp.int32)

    gd = goal.shape[1]
    kcat = f_in + gd
    kpad = ((kcat + 7) // 8) * 8
    xg = jnp.pad(jnp.concatenate([x, goal], axis=1), ((0, pad), (0, kpad - kcat)))
    w1 = jnp.pad(params['W1'], ((0, kpad - kcat), (0, 0)))
    ew = _gate(xg, w1, params['b1'], params['W2'], params['b2'], n)

    h1 = _one_conv(xp, src_s, dst_s, rowptr, params['c1'], False)
    h2 = _one_conv(h1, src_s, dst_s, rowptr, params['c2'], True)
    h3 = _one_conv(h2, src_s, dst_s, rowptr, params['c3'], True)

    batch2d = jnp.pad(batch.astype(jnp.int32), (0, pad)).reshape(NP, 1)
    pooled = _pool(h3, ew, batch2d, ng)
    return _final(pooled, params['Wm1'], params['bm1'], params['Wm2'], params['bm2'])


# speedup vs baseline: 3.5863x; 1.1179x over previous
import functools
import jax
import jax.numpy as jnp
from jax import lax
from jax.experimental import pallas as pl
from jax.experimental.pallas import tpu as pltpu
from jax.experimental.pallas import tpu_sc as plsc

H = 2
BR = 512
NC = 2
NS = 16
NW = NC * NS
NP = 51200
WIN = NP // NW
GRP = 64
NGRP = WIN // GRP
EC = 128


def _proj_body(x_ref, w_ref, b_ref, q_ref, kv_ref):
    m = q_ref.shape[1]
    o = jnp.dot(x_ref[...], w_ref[...],
                preferred_element_type=jnp.float32) + b_ref[...]
    q_ref[...] = o[:, :m]
    kv_ref[...] = o[:, m:]


def _proj3(x, w, b):
    np_, k = x.shape
    m3 = w.shape[1]
    m = m3 // 3
    return pl.pallas_call(
        _proj_body,
        grid=(np_ // BR,),
        in_specs=[
            pl.BlockSpec((BR, k), lambda i: (i, 0)),
            pl.BlockSpec((k, m3), lambda i: (0, 0)),
            pl.BlockSpec((1, m3), lambda i: (0, 0)),
        ],
        out_specs=[pl.BlockSpec((BR, m), lambda i: (i, 0)),
                   pl.BlockSpec((BR, 2 * m), lambda i: (i, 0))],
        out_shape=[jax.ShapeDtypeStruct((np_, m), jnp.float32),
                   jax.ShapeDtypeStruct((np_, 2 * m), jnp.float32)],
    )(x, w, b.reshape(1, m3))


def _epi_body(raw_ref, s0_ref, s1_ref, xs_ref, wa_ref, ba_ref, beta_ref,
              o_ref, *, d):
    s0, s1 = s0_ref[...], s1_ref[...]
    raw = raw_ref[...]
    m0 = jnp.where(s0 > 0, raw[:, :d] / (s0 + 1e-16), 0.0)
    m1 = jnp.where(s1 > 0, raw[:, d:] / (s1 + 1e-16), 0.0)
    msg = jnp.concatenate([m0, m1], axis=1)
    out = jnp.dot(jax.nn.gelu(msg), wa_ref[...],
                  preferred_element_type=jnp.float32) + ba_ref[...]
    beta = beta_ref[0, 0]
    o_ref[...] = jnp.maximum(beta * out + (1.0 - beta) * xs_ref[...], 0.0)


def _conv_epi(raw, s0, s1, x_skip, wa, ba, beta):
    np_, cout = raw.shape
    return pl.pallas_call(
        functools.partial(_epi_body, d=cout // H),
        grid=(np_ // BR,),
        in_specs=[
            pl.BlockSpec((BR, cout), lambda i: (i, 0)),
            pl.BlockSpec((BR, 1), lambda i: (i, 0)),
            pl.BlockSpec((BR, 1), lambda i: (i, 0)),
            pl.BlockSpec((BR, cout), lambda i: (i, 0)),
            pl.BlockSpec((cout, cout), lambda i: (0, 0)),
            pl.BlockSpec((1, cout), lambda i: (0, 0)),
            pl.BlockSpec((1, 1), lambda i: (0, 0), memory_space=pltpu.SMEM),
        ],
        out_specs=pl.BlockSpec((BR, cout), lambda i: (i, 0)),
        out_shape=jax.ShapeDtypeStruct((np_, cout), jnp.float32),
    )(raw, s0.reshape(np_, 1), s1.reshape(np_, 1), x_skip,
      wa, ba.reshape(1, cout), beta.reshape(1, 1))


def _gate_body(xg_ref, w1_ref, b1_ref, w2_ref, b2_ref, o_ref, *, n_real):
    i = pl.program_id(0)
    h = jnp.maximum(jnp.dot(xg_ref[...], w1_ref[...],
                            preferred_element_type=jnp.float32) + b1_ref[...], 0.0)
    w = jnp.dot(h, w2_ref[...], preferred_element_type=jnp.float32) + b2_ref[...]
    row = i * BR + jax.lax.broadcasted_iota(jnp.int32, w.shape, 0)
    o_ref[...] = jnp.where(row < n_real, jnp.exp(w), 0.0)


def _gate(xg, w1, b1, w2, b2, n_real):
    np_, k = xg.shape
    return pl.pallas_call(
        functools.partial(_gate_body, n_real=n_real),
        grid=(np_ // BR,),
        in_specs=[
            pl.BlockSpec((BR, k), lambda i: (i, 0)),
            pl.BlockSpec((k, 32), lambda i: (0, 0)),
            pl.BlockSpec((1, 32), lambda i: (0, 0)),
            pl.BlockSpec((32, 1), lambda i: (0, 0)),
            pl.BlockSpec((1, 1), lambda i: (0, 0)),
        ],
        out_specs=pl.BlockSpec((BR, 1), lambda i: (i, 0)),
        out_shape=jax.ShapeDtypeStruct((np_, 1), jnp.float32),
    )(xg, w1, b1.reshape(1, 32), w2, b2.reshape(1, 1))


def _pool_body(h3_ref, ew_ref, b_ref, o_ref, *, ng):
    i = pl.program_id(0)

    @pl.when(i == 0)
    def _():
        o_ref[...] = jnp.zeros_like(o_ref)

    ew = ew_ref[...]
    feats = jnp.concatenate([h3_ref[...] * ew, ew], axis=1)
    oh = (b_ref[...] == jax.lax.broadcasted_iota(jnp.int32, (BR, ng), 1)
          ).astype(jnp.float32)
    o_ref[...] += jax.lax.dot_general(
        oh, feats, (((0,), (0,)), ((), ())),
        preferred_element_type=jnp.float32)


def _pool(h3, ew, batch2d, ng):
    np_, cout = h3.shape
    return pl.pallas_call(
        functools.partial(_pool_body, ng=ng),
        grid=(np_ // BR,),
        in_specs=[
            pl.BlockSpec((BR, cout), lambda i: (i, 0)),
            pl.BlockSpec((BR, 1), lambda i: (i, 0)),
            pl.BlockSpec((BR, 1), lambda i: (i, 0)),
        ],
        out_specs=pl.BlockSpec((ng, cout + 1), lambda i: (0, 0)),
        out_shape=jax.ShapeDtypeStruct((ng, cout + 1), jnp.float32),
    )(h3, ew, batch2d)


def _final_body(p_ref, wm1_ref, bm1_ref, wm2_ref, bm2_ref, o_ref, *, cout):
    p = p_ref[...]
    g = p[:, :cout] / (p[:, cout:cout + 1] + 1e-16)
    h = jnp.maximum(jnp.dot(g, wm1_ref[...],
                            preferred_element_type=jnp.float32) + bm1_ref[...], 0.0)
    o_ref[...] = jnp.dot(h, wm2_ref[...],
                         preferred_element_type=jnp.float32) + bm2_ref[...]


def _final(pooled, wm1, bm1, wm2, bm2):
    ng, c1 = pooled.shape
    cout = c1 - 1
    hid = wm1.shape[1]
    out = wm2.shape[1]
    return pl.pallas_call(
        functools.partial(_final_body, cout=cout),
        in_specs=[
            pl.BlockSpec((ng, c1), lambda: (0, 0)),
            pl.BlockSpec((cout, hid), lambda: (0, 0)),
            pl.BlockSpec((1, hid), lambda: (0, 0)),
            pl.BlockSpec((hid, out), lambda: (0, 0)),
            pl.BlockSpec((1, out), lambda: (0, 0)),
        ],
        out_specs=pl.BlockSpec((ng, out), lambda: (0, 0)),
        out_shape=jax.ShapeDtypeStruct((ng, out), jnp.float32),
    )(pooled, wm1, bm1.reshape(1, hid), wm2, bm2.reshape(1, out))


def _zero16(ref, r, c0, nchunks):
    z = jnp.zeros((16,), jnp.float32)
    for c in range(nchunks):
        ref[r, pl.ds(c0 + 16 * c, 16)] = z


def _sload(ref, i):
    return ref[pl.ds(i, 16)][0]


_GDN = jax.lax.GatherDimensionNumbers(
    offset_dims=(), collapsed_slice_dims=(0,), start_index_map=(0,))


def _bfly_sum(x):
    for k in (1, 2, 4, 8):
        perm = jnp.bitwise_xor(lax.iota(jnp.int32, 16), k)
        x = x + jax.lax.gather(x, perm[:, None], _GDN, (1,),
                               mode=jax.lax.GatherScatterMode.PROMISE_IN_BOUNDS)
    return x


def _edge_sc_body(q_hbm, kv_hbm, src_hbm, dst_hbm, rp_hbm,
                  raw_hbm, s0_hbm, s1_hbm,
                  rp_v, q_g, kvrA, kvrB, srcA, srcB, dstA, dstB,
                  st, s0_st, s1_st, s0_grp, s1_grp, e0_buf, e1_buf,
                  semA, semB):
    wid = lax.axis_index("s") * NC + lax.axis_index("c")
    v0 = wid * WIN
    pltpu.sync_copy(rp_hbm.at[pl.ds(v0, WIN + 16)], rp_v)

    def group_body(g, _):
        vg = v0 + g * GRP
        pltpu.sync_copy(q_hbm.at[pl.ds(vg, GRP)], q_g)

        def zrow(r, _):
            _zero16(st, r, 0, 8)
            s0_st[r, :] = jnp.zeros((16,), jnp.float32)
            s1_st[r, :] = jnp.zeros((16,), jnp.float32)
            return 0
        lax.fori_loop(0, GRP, zrow, 0)

        e_start = _sload(rp_v, g * GRP)
        e_end = _sload(rp_v, (g + 1) * GRP)
        ca0 = (e_start // 8) * 8
        nch = (e_end - ca0 + EC - 1) // EC

        def load_and_issue(c, src_c, dst_c, kvr, sem):
            ca = ca0 + c * EC
            pltpu.sync_copy(src_hbm.at[pl.ds(ca, EC)], src_c)
            pltpu.sync_copy(dst_hbm.at[pl.ds(ca, EC + 16)], dst_c)
            pltpu.make_async_copy(kv_hbm.at[src_c], kvr, sem).start()

        def compute(c, dst_c, kvr):
            ca = ca0 + c * EC
            lo = jnp.maximum(e_start - ca, 0)
            hi = jnp.minimum(e_end - ca, EC)

            def edge_body(j, _):
                n = _sload(dst_c, j) - vg
                acc0 = q_g[n, pl.ds(0, 16)] * kvr[j, pl.ds(0, 16)]
                acc1 = q_g[n, pl.ds(64, 16)] * kvr[j, pl.ds(64, 16)]
                for c4 in range(1, 4):
                    acc0 += q_g[n, pl.ds(16 * c4, 16)] * kvr[j, pl.ds(16 * c4, 16)]
                    acc1 += (q_g[n, pl.ds(64 + 16 * c4, 16)]
                             * kvr[j, pl.ds(64 + 16 * c4, 16)])
                e0 = jnp.exp(_bfly_sum(acc0))
                e1 = jnp.exp(_bfly_sum(acc1))
                for c4 in range(4):
                    plsc.addupdate(st.at[n, pl.ds(16 * c4, 16)],
                                   e0 * kvr[j, pl.ds(128 + 16 * c4, 16)])
                    plsc.addupdate(st.at[n, pl.ds(64 + 16 * c4, 16)],
                                   e1 * kvr[j, pl.ds(192 + 16 * c4, 16)])
                plsc.addupdate(s0_st.at[n], e0)
                plsc.addupdate(s1_st.at[n], e1)
                return 0
            lax.fori_loop(lo, hi, edge_body, 0)

        def waitA():
            pltpu.make_async_copy(kv_hbm.at[srcA], kvrA, semA).wait()

        def waitB():
            pltpu.make_async_copy(kv_hbm.at[srcB], kvrB, semB).wait()

        @pl.when(nch > 0)
        def _():
            load_and_issue(0, srcA, dstA, kvrA, semA)

        def pair_body(kk, _):
            c0 = 2 * kk
            c1 = c0 + 1
            waitA()

            @pl.when(c1 < nch)
            def _():
                load_and_issue(c1, srcB, dstB, kvrB, semB)
            compute(c0, dstA, kvrA)

            @pl.when(c1 < nch)
            def _():
                waitB()

            @pl.when(c1 + 1 < nch)
            def _():
                load_and_issue(c1 + 1, srcA, dstA, kvrA, semA)

            @pl.when(c1 < nch)
            def _():
                compute(c1, dstB, kvrB)
            return 0
        lax.fori_loop(0, (nch + 1) // 2, pair_body, 0)

        rows16 = lax.iota(jnp.int32, 16)
        zc = jnp.zeros((16,), jnp.int32)
        for r in range(4):
            idx = rows16 + 16 * r
            s0_grp[pl.ds(16 * r, 16)] = plsc.load_gather(s0_st, [idx, zc])
            s1_grp[pl.ds(16 * r, 16)] = plsc.load_gather(s1_st, [idx, zc])
        pltpu.sync_copy(st, raw_hbm.at[pl.ds(vg, GRP)])
        pltpu.sync_copy(s0_grp, s0_hbm.at[pl.ds(vg, GRP)])
        pltpu.sync_copy(s1_grp, s1_hbm.at[pl.ds(vg, GRP)])
        return 0
    lax.fori_loop(0, NGRP, group_body, 0)


def _edge_phase(q, kv, src_s, dst_s, rowptr):
    mesh = plsc.VectorSubcoreMesh(core_axis_name="c", subcore_axis_name="s")
    f = pl.kernel(
        _edge_sc_body,
        mesh=mesh,
        compiler_params=pltpu.CompilerParams(needs_layout_passes=False),
        out_type=[
            jax.ShapeDtypeStruct((NP, H * 64), jnp.float32),
            jax.ShapeDtypeStruct((NP,), jnp.float32),
            jax.ShapeDtypeStruct((NP,), jnp.float32),
        ],
        scratch_types=[
            pltpu.VMEM((WIN + 16,), jnp.int32),
            pltpu.VMEM((GRP, H * 64), jnp.float32),
            pltpu.VMEM((EC, 2 * H * 64), jnp.float32),
            pltpu.VMEM((EC, 2 * H * 64), jnp.float32),
            pltpu.VMEM((EC,), jnp.int32),
            pltpu.VMEM((EC,), jnp.int32),
            pltpu.VMEM((EC + 16,), jnp.int32),
            pltpu.VMEM((EC + 16,), jnp.int32),
            pltpu.VMEM((GRP, H * 64), jnp.float32),
            pltpu.VMEM((GRP, 16), jnp.float32),
            pltpu.VMEM((GRP, 16), jnp.float32),
            pltpu.VMEM((GRP,), jnp.float32),
            pltpu.VMEM((GRP,), jnp.float32),
            pltpu.VMEM((EC + 16,), jnp.float32),
            pltpu.VMEM((EC + 16,), jnp.float32),
            pltpu.SemaphoreType.DMA,
            pltpu.SemaphoreType.DMA,
        ],
    )
    return f(q, kv, src_s, dst_s, rowptr)


def _fold_kv(p, cin):
    cout = p['Wk'].shape[1]
    d = cout // H
    wk = jnp.einsum('chd,hde->che', p['Wk'].reshape(cin, H, d), p['a_rel']).reshape(cin, cout)
    bk = jnp.einsum('hd,hde->he', p['bk'].reshape(H, d), p['a_rel']).reshape(cout)
    wv = jnp.einsum('chd,hde->che', p['Wv'].reshape(cin, H, d), p['m_rel']).reshape(cin, cout)
    bv = jnp.einsum('hd,hde->he', p['bv'].reshape(H, d), p['m_rel']).reshape(cout)
    return wk, bk, wv, bv


def _one_conv(x_in, src_s, dst_s, rowptr, p, use_skip):
    cin = x_in.shape[1]
    cout = p['Wk'].shape[1]
    d = cout // H
    wk, bk, wv, bv = _fold_kv(p, cin)
    scale = p['p_rel'] / jnp.sqrt(jnp.float32(d))
    wq = p['Wq'] * jnp.repeat(scale, d)[None, :]
    bq = p['bq'] * jnp.repeat(scale, d)
    wqkv = jnp.concatenate([wq, wk, wv], axis=1)
    bqkv = jnp.concatenate([bq, bk, bv])
    q, kv = _proj3(x_in, wqkv, bqkv)
    raw, s0, s1 = _edge_phase(q, kv, src_s, dst_s, rowptr)
    beta = jax.nn.sigmoid(p['skip']) if use_skip else jnp.float32(1.0)
    x_skip = x_in if use_skip else raw
    return _conv_epi(raw, s0, s1, x_skip, p['Wa'], p['ba'], beta)


def kernel(x, goal, edge_index, batch, params):
    n, f_in = x.shape
    ng = 64
    pad = NP - n
    xp = jnp.pad(x, ((0, pad), (0, 0)))

    src, dst = edge_index[0].astype(jnp.int32), edge_index[1].astype(jnp.int32)
    dst_sorted, src_sorted = jax.lax.sort_key_val(dst, src)
    src_s = jnp.pad(src_sorted, (0, 2 * EC))
    dst_s = jnp.pad(dst_sorted, (0, 2 * EC))
    rowptr = jnp.searchsorted(dst_sorted,
                              jnp.arange(NP + 16, dtype=jnp.int32)).astype(jnp.int32)

    gd = goal.shape[1]
    kcat = f_in + gd
    kpad = ((kcat + 7) // 8) * 8
    xg = jnp.pad(jnp.concatenate([x, goal], axis=1), ((0, pad), (0, kpad - kcat)))
    w1 = jnp.pad(params['W1'], ((0, kpad - kcat), (0, 0)))
    ew = _gate(xg, w1, params['b1'], params['W2'], params['b2'], n)

    h1 = _one_conv(xp, src_s, dst_s, rowptr, params['c1'], False)
    h2 = _one_conv(h1, src_s, dst_s, rowptr, params['c2'], True)
    h3 = _one_conv(h2, src_s, dst_s, rowptr, params['c3'], True)

    batch2d = jnp.pad(batch.astype(jnp.int32), (0, pad)).reshape(NP, 1)
    pooled = _pool(h3, ew, batch2d, ng)
    return _final(pooled, params['Wm1'], params['bm1'], params['Wm2'], params['bm2'])
